# Initial kernel scaffold; baseline (speedup 1.0000x reference)
import jax
import jax.numpy as jnp
from jax.experimental import pallas as pl


def kernel(x, edge_index_rel0, edge_index_rel1, W0, b0, W1, b1, prelu_a):
    raise NotImplementedError("write your pallas kernel here")


# speedup vs baseline: 8.9873x; 8.9873x over previous
import jax
import jax.numpy as jnp
from jax import lax
from jax.experimental import pallas as pl
from jax.experimental.pallas import tpu as pltpu
from jax.experimental.pallas import tpu_sc as plsc

N = 10000
E = 320000
D = 128
DH = D // 2
NS = 16
NC = 2
NPAD = 10240
RPT = NPAD // NS
EC = E // NS
BLK = 128
NB = (EC + BLK - 1) // BLK
ECP = NB * BLK
CH = 32
NCH = RPT // CH


def _rsqrt16(d):
    xi = plsc.bitcast(d, jnp.int32)
    yi = jnp.int32(0x5F3759DF) - (xi >> 1)
    y = plsc.bitcast(yi, jnp.float32)
    for _ in range(3):
        y = y * (1.5 - 0.5 * d * y * y)
    return y


def _sc_body(x2_hbm, pk_hbm, xs_hbm, p_hbm,
             pk, idx_src, idx_dst, rows, xrow, nsrc, ndst, z128, zrow, ones,
             acc_sh, degs_sh, degd_sh, sem):
    rel = lax.axis_index("c")
    tile = lax.axis_index("s")
    arow0 = tile * RPT

    z16 = jnp.zeros((16,), jnp.float32)
    for i in range(8):
        z128[pl.ds(i * 16, 16)] = z16
        ones[pl.ds(i * 16, 16)] = z16 + 1.0
    for i in range(8):
        for k in range(DH // 16):
            zrow[i, pl.ds(k * 16, 16)] = z16

    def zero_deg(i, _):
        pltpu.sync_copy(z128, degs_sh.at[pl.ds(rel * NPAD + arow0 + 128 * i, 128)])
        pltpu.sync_copy(z128, degd_sh.at[pl.ds(arow0 + 128 * i, 128)])
        return _
    lax.fori_loop(0, RPT // 128, zero_deg, None)

    plsc.subcore_barrier()

    pltpu.sync_copy(pk_hbm.at[rel, tile], pk)

    def unpack_blk(j, _):
        for k in range(BLK // 16):
            s = pl.ds(k * 16, 16)
            v = pk[j, s]
            idx_src[j, s] = v >> 14
            idx_dst[j, s] = v & 16383
        return _
    lax.fori_loop(0, NB, unpack_blk, None)

    def deg_blk(j, _):
        pltpu.sync_copy(ones, degs_sh.at[idx_src.at[j]], add=True)
        pltpu.sync_copy(ones, degd_sh.at[idx_dst.at[j]], add=True)
        return _
    lax.fori_loop(0, NB, deg_blk, None)

    plsc.subcore_barrier()

    pltpu.sync_copy(degs_sh.at[pl.ds(rel * NPAD + arow0, RPT)], nsrc)
    pltpu.sync_copy(degd_sh.at[pl.ds(arow0, RPT)], ndst)

    def norm_blk(i, _):
        s = pl.ds(i * 16, 16)
        nsrc[s] = _rsqrt16(jnp.maximum(nsrc[s], 1.0))
        ndst[s] = _rsqrt16(jnp.maximum(ndst[s], 1.0))
        return _
    lax.fori_loop(0, RPT // 16, norm_blk, None)

    for fh in range(2):
        def stage_chunk(c, _):
            r0 = c * CH
            pltpu.sync_copy(x2_hbm.at[pl.ds(fh * NPAD + arow0 + r0, CH)], xrow)
            for i in range(CH):
                w = plsc.load_gather(nsrc, [jnp.full((16,), r0 + i, jnp.int32)])
                for k in range(DH // 16):
                    s = pl.ds(k * 16, 16)
                    xrow[i, s] = xrow[i, s] * w
            pltpu.sync_copy(
                xrow,
                xs_hbm.at[pl.ds((2 * fh + rel) * NPAD + arow0 + r0, CH)])
            return _
        lax.fori_loop(0, NCH, stage_chunk, None)

    plsc.subcore_barrier()

    for fh in range(2):
        def zero_acc(i, _):
            pltpu.sync_copy(zrow, acc_sh.at[pl.ds(arow0 + 8 * i, 8)])
            return _
        lax.fori_loop(0, RPT // 8, zero_acc, None)

        if fh == 1:
            def shift_blk(j, _):
                for k in range(BLK // 16):
                    s = pl.ds(k * 16, 16)
                    idx_src[j, s] = idx_src[j, s] + 2 * NPAD
                return _
            lax.fori_loop(0, NB, shift_blk, None)

        plsc.subcore_barrier()

        pltpu.async_copy(xs_hbm.at[idx_src.at[0]], rows.at[0], sem)

        def edge_blk(j, _):
            buf = lax.rem(j, 2)
            @pl.when(j < NB - 1)
            def _issue():
                pltpu.async_copy(
                    xs_hbm.at[idx_src.at[j + 1]], rows.at[1 - buf], sem)
            pltpu.make_async_copy(
                xs_hbm.at[idx_src.at[j]], rows.at[buf], sem).wait()
            pltpu.sync_copy(rows.at[buf], acc_sh.at[idx_dst.at[j]], add=True)
            return _
        lax.fori_loop(0, NB, edge_blk, None)

        plsc.subcore_barrier()

        def out_chunk(c, _):
            r0 = c * CH
            pltpu.sync_copy(acc_sh.at[pl.ds(arow0 + r0, CH)], xrow)
            for i in range(CH):
                w = plsc.load_gather(ndst, [jnp.full((16,), r0 + i, jnp.int32)])
                for k in range(DH // 16):
                    s = pl.ds(k * 16, 16)
                    xrow[i, s] = xrow[i, s] * w
            pltpu.sync_copy(
                xrow,
                p_hbm.at[pl.ds((2 * fh + rel) * NPAD + arow0 + r0, CH)])
            return _
        lax.fori_loop(0, NCH, out_chunk, None)


def _tc_body(p0, p1, w0, w1, b0, b1, a, o):
    h = jnp.dot(p0[...], w0[...], preferred_element_type=jnp.float32,
                precision=lax.Precision.HIGHEST)
    h = h + jnp.dot(p1[...], w1[...], preferred_element_type=jnp.float32,
                    precision=lax.Precision.HIGHEST)
    h = h + b0[...] + b1[...]
    o[...] = jnp.where(h > 0, h, a[0, 0] * h)


def _pad_edges(src, dst, rel):
    src = src.reshape(NS, EC)
    dst = dst.reshape(NS, EC)
    npad = ECP - EC
    k = jnp.arange(npad, dtype=jnp.int32)[None, :] + 17 * jnp.arange(
        NS, dtype=jnp.int32)[:, None]
    padv = N + ((k * 7) % (NPAD - N))
    src = jnp.concatenate([src, padv], axis=1)
    dst = jnp.concatenate([dst, padv], axis=1)
    src = src + rel * NPAD
    return ((src << 14) | dst).reshape(NS, NB, BLK)


@jax.jit
def kernel(x, edge_index_rel0, edge_index_rel1, W0, b0, W1, b1, prelu_a):
    x_pad = jnp.zeros((NPAD, D), jnp.float32).at[:N].set(x)
    x2 = jnp.concatenate([x_pad[:, :DH], x_pad[:, DH:]], axis=0)
    pk0 = _pad_edges(edge_index_rel0[0], edge_index_rel0[1], 0)
    pk1 = _pad_edges(edge_index_rel1[0], edge_index_rel1[1], 1)
    pk_all = jnp.stack([pk0, pk1])

    mesh = plsc.VectorSubcoreMesh(core_axis_name="c", subcore_axis_name="s")
    sc = pl.kernel(
        _sc_body,
        out_type=(
            jax.ShapeDtypeStruct((4 * NPAD, DH), jnp.float32),
            jax.ShapeDtypeStruct((4 * NPAD, DH), jnp.float32),
        ),
        mesh=mesh,
        compiler_params=pltpu.CompilerParams(
            needs_layout_passes=False, use_tc_tiling_on_sc=False),
        scratch_types=[
            pltpu.VMEM((NB, BLK), jnp.int32),
            pltpu.VMEM((NB, BLK), jnp.int32),
            pltpu.VMEM((NB, BLK), jnp.int32),
            pltpu.VMEM((2, BLK, DH), jnp.float32),
            pltpu.VMEM((CH, DH), jnp.float32),
            pltpu.VMEM((RPT,), jnp.float32),
            pltpu.VMEM((RPT,), jnp.float32),
            pltpu.VMEM((128,), jnp.float32),
            pltpu.VMEM((8, DH), jnp.float32),
            pltpu.VMEM((BLK,), jnp.float32),
            pltpu.VMEM_SHARED((NPAD, DH), jnp.float32),
            pltpu.VMEM_SHARED((NC * NPAD,), jnp.float32),
            pltpu.VMEM_SHARED((NPAD,), jnp.float32),
            pltpu.SemaphoreType.DMA,
        ],
    )
    _, p = sc(x2, pk_all)

    p0 = jnp.concatenate([p[:N], p[2 * NPAD:2 * NPAD + N]], axis=1)
    p1 = jnp.concatenate([p[NPAD:NPAD + N], p[3 * NPAD:3 * NPAD + N]], axis=1)
    bs = 400
    h = pl.pallas_call(
        _tc_body,
        grid=(N // bs,),
        in_specs=[
            pl.BlockSpec((bs, D), lambda j: (j, 0)),
            pl.BlockSpec((bs, D), lambda j: (j, 0)),
            pl.BlockSpec((D, D), lambda j: (0, 0)),
            pl.BlockSpec((D, D), lambda j: (0, 0)),
            pl.BlockSpec((1, D), lambda j: (0, 0)),
            pl.BlockSpec((1, D), lambda j: (0, 0)),
            pl.BlockSpec((1, 1), lambda j: (0, 0)),
        ],
        out_specs=pl.BlockSpec((bs, D), lambda j: (j, 0)),
        out_shape=jax.ShapeDtypeStruct((N, D), jnp.float32),
    )(p0, p1, W0, W1, b0.reshape(1, D), b1.reshape(1, D),
      prelu_a.reshape(1, 1))
    return h
